# baseline (speedup 1.0000x reference)
import functools

import jax
import jax.numpy as jnp
from jax import lax
from jax.experimental import pallas as pl
from jax.experimental.pallas import tpu as pltpu
from jax.experimental.pallas import tpu_sc as plsc

_NC = 2
_NS = 16
_NW = _NC * _NS

_DIM = 32
_BATCH = 16384
_HIST = 200
_B = _BATCH * _HIST

_CHUNK = 512
_SPB = _BATCH // _CHUNK
_NUNIT = _HIST * _SPB
_PER_W = _NUNIT // _NW
_TPC = _CHUNK // 128


def _make_kernel():
  mesh = plsc.VectorSubcoreMesh(
      core_axis_name="c", subcore_axis_name="s",
      num_cores=_NC, num_subcores=_NS)

  @functools.partial(
      pl.kernel,
      out_type=jax.ShapeDtypeStruct((_B * _DIM,), jnp.float32),
      mesh=mesh,
      scratch_types=(
          [pltpu.VMEM((_CHUNK,), jnp.int32) for _ in range(2)]
          + [pltpu.VMEM((_CHUNK, _DIM), jnp.float32) for _ in range(2)]
          + [pltpu.VMEM((_CHUNK * _DIM,), jnp.float32) for _ in range(2)]
          + [pltpu.SemaphoreType.DMA] * 3
      ),
      compiler_params=pltpu.CompilerParams(
          use_tc_tiling_on_sc=False, needs_layout_passes=False),
  )
  def gather_kernel(xt_hbm, table_hbm, out_hbm, idx0, idx1, rows0, rows1,
                    tr0, tr1, sem_i, sem_g, sem_s):
    idx_bufs = (idx0, idx1)
    row_bufs = (rows0, rows1)
    tr_bufs = (tr0, tr1)

    wid = lax.axis_index("s") * _NC + lax.axis_index("c")
    base_u = wid * _PER_W
    lane = lax.iota(jnp.int32, 16)
    offv_lo = (lane >> 3) * (_TPC * 1024) + (lane & 7) * 128
    offv_hi = offv_lo + 2 * (_TPC * 1024)

    def unit_hs(i):
      u = base_u + i
      return u // _SPB, u % _SPB

    def issue_idx(i, p):
      h, s = unit_hs(i)
      pltpu.async_copy(
          xt_hbm.at[pl.ds(h * _BATCH + s * _CHUNK, _CHUNK)],
          idx_bufs[p], sem_i)

    def issue_gather(p):
      pltpu.async_copy(table_hbm.at[idx_bufs[p]], row_bufs[p], sem_g)

    def issue_store(i, p):
      h, s = unit_hs(i)
      for dt in range(_DIM // 8):
        off = ((h * (_DIM // 8) + dt) * (_BATCH // 128) + s * _TPC) * 1024
        pltpu.async_copy(
            tr_bufs[p].at[pl.ds(dt * (_TPC * 1024), _TPC * 1024)],
            out_hbm.at[pl.ds(off, _TPC * 1024)], sem_s)

    def wait_idx():
      pltpu.make_async_copy(
          xt_hbm.at[pl.ds(0, _CHUNK)], idx_bufs[0], sem_i).wait()

    def wait_gather():
      pltpu.make_async_copy(
          table_hbm.at[idx_bufs[0]], row_bufs[0], sem_g).wait()

    def wait_store():
      for dt in range(_DIM // 8):
        pltpu.make_async_copy(
            tr_bufs[0].at[pl.ds(0, _TPC * 1024)],
            out_hbm.at[pl.ds(0, _TPC * 1024)], sem_s).wait()

    def transpose_unit(p):
      rows = row_bufs[p]
      tr = tr_bufs[p]

      def b_body(b, carry):
        s_off = (b >> 7) * 1024 + (b & 127)
        v0 = rows[b, pl.ds(0, 16)]
        v1 = rows[b, pl.ds(16, 16)]
        plsc.store_scatter(tr, [offv_lo + s_off], v0)
        plsc.store_scatter(tr, [offv_hi + s_off], v1)
        return carry

      lax.fori_loop(0, _CHUNK, b_body, 0)

    issue_idx(0, 0)
    issue_idx(1, 1)
    wait_idx()
    issue_gather(0)

    def round_body(r, carry):
      for p in range(2):
        i = 2 * r + p
        @pl.when(i + 1 < _PER_W)
        def _():
          wait_idx()
          issue_gather(1 - p)

        wait_gather()

        @pl.when(i >= 2)
        def _():
          wait_store()

        transpose_unit(p)
        issue_store(i, p)

        @pl.when(i + 2 < _PER_W)
        def _():
          issue_idx(i + 2, p)
      return carry

    lax.fori_loop(0, _PER_W // 2, round_body, 0)
    wait_store()
    wait_store()

  return gather_kernel


_GATHER = _make_kernel()


def kernel(x, table):
  xt = jnp.transpose(x).reshape(-1).astype(jnp.int32)
  u = _GATHER(xt, table)
  u5 = u.reshape(_HIST, _DIM // 8, _BATCH // 128, 8, 128)
  v = jnp.transpose(u5, (0, 1, 3, 2, 4)).reshape(_HIST, _DIM, _BATCH)
  return jnp.transpose(v, (2, 0, 1))


# speedup vs baseline: 1.1287x; 1.1287x over previous
import functools

import jax
import jax.numpy as jnp
from jax import lax
from jax.experimental import pallas as pl
from jax.experimental.pallas import tpu as pltpu
from jax.experimental.pallas import tpu_sc as plsc

_NC = 2
_NS = 16
_NW = _NC * _NS

_DIM = 32
_BATCH = 16384
_HIST = 200
_B = _BATCH * _HIST

_CHUNK = 512
_SPB = _BATCH // _CHUNK
_NUNIT = _HIST * _SPB
_PER_W = _NUNIT // _NW
_TPC = _CHUNK // 128


def _make_kernel():
  mesh = plsc.VectorSubcoreMesh(
      core_axis_name="c", subcore_axis_name="s",
      num_cores=_NC, num_subcores=_NS)

  @functools.partial(
      pl.kernel,
      out_type=jax.ShapeDtypeStruct((_B * _DIM,), jnp.float32),
      mesh=mesh,
      scratch_types=(
          [pltpu.VMEM((_CHUNK,), jnp.int32) for _ in range(2)]
          + [pltpu.VMEM((_CHUNK, _DIM), jnp.float32) for _ in range(2)]
          + [pltpu.VMEM((_CHUNK * _DIM,), jnp.float32) for _ in range(2)]
          + [pltpu.SemaphoreType.DMA] * 3
      ),
      compiler_params=pltpu.CompilerParams(
          use_tc_tiling_on_sc=False, needs_layout_passes=False),
  )
  def gather_kernel(xt_hbm, table_hbm, out_hbm, idx0, idx1, rows0, rows1,
                    tr0, tr1, sem_i, sem_g, sem_s):
    idx_bufs = (idx0, idx1)
    row_bufs = (rows0, rows1)
    tr_bufs = (tr0, tr1)

    wid = lax.axis_index("s") * _NC + lax.axis_index("c")
    base_u = wid * _PER_W
    lane = lax.iota(jnp.int32, 16)
    offv_lo = (lane >> 3) * (_TPC * 1024) + (lane & 7) * 128
    offv_hi = offv_lo + 2 * (_TPC * 1024)

    def unit_hs(i):
      u = base_u + i
      return u // _SPB, u % _SPB

    def issue_idx(i, p):
      h, s = unit_hs(i)
      pltpu.async_copy(
          xt_hbm.at[pl.ds(h * _BATCH + s * _CHUNK, _CHUNK)],
          idx_bufs[p], sem_i)

    def issue_gather(p):
      pltpu.async_copy(table_hbm.at[idx_bufs[p]], row_bufs[p], sem_g)

    def issue_store(i, p):
      h, s = unit_hs(i)
      for dt in range(_DIM // 8):
        off = ((h * (_DIM // 8) + dt) * (_BATCH // 128) + s * _TPC) * 1024
        pltpu.async_copy(
            tr_bufs[p].at[pl.ds(dt * (_TPC * 1024), _TPC * 1024)],
            out_hbm.at[pl.ds(off, _TPC * 1024)], sem_s)

    def wait_idx():
      pltpu.make_async_copy(
          xt_hbm.at[pl.ds(0, _CHUNK)], idx_bufs[0], sem_i).wait()

    def wait_gather():
      pltpu.make_async_copy(
          table_hbm.at[idx_bufs[0]], row_bufs[0], sem_g).wait()

    def wait_store():
      for dt in range(_DIM // 8):
        pltpu.make_async_copy(
            tr_bufs[0].at[pl.ds(0, _TPC * 1024)],
            out_hbm.at[pl.ds(0, _TPC * 1024)], sem_s).wait()

    def transpose_unit(p):
      rows = row_bufs[p]
      tr = tr_bufs[p]

      @plsc.parallel_loop(0, _CHUNK, 1, unroll=8)
      def _(b):
        s_off = (b >> 7) * 1024 + (b & 127)
        v0 = rows[b, pl.ds(0, 16)]
        v1 = rows[b, pl.ds(16, 16)]
        plsc.store_scatter(tr, [offv_lo + s_off], v0)
        plsc.store_scatter(tr, [offv_hi + s_off], v1)

    issue_idx(0, 0)
    issue_idx(1, 1)
    wait_idx()
    issue_gather(0)

    def round_body(r, carry):
      for p in range(2):
        i = 2 * r + p
        @pl.when(i + 1 < _PER_W)
        def _():
          wait_idx()
          issue_gather(1 - p)

        wait_gather()

        @pl.when(i >= 2)
        def _():
          wait_store()

        transpose_unit(p)
        issue_store(i, p)

        @pl.when(i + 2 < _PER_W)
        def _():
          issue_idx(i + 2, p)
      return carry

    lax.fori_loop(0, _PER_W // 2, round_body, 0)
    wait_store()
    wait_store()

  return gather_kernel


_GATHER = _make_kernel()


def kernel(x, table):
  xt = jnp.transpose(x).reshape(-1).astype(jnp.int32)
  u = _GATHER(xt, table)
  u5 = u.reshape(_HIST, _DIM // 8, _BATCH // 128, 8, 128)
  v = jnp.transpose(u5, (0, 1, 3, 2, 4)).reshape(_HIST, _DIM, _BATCH)
  return jnp.transpose(v, (2, 0, 1))


# speedup vs baseline: 1.3284x; 1.1769x over previous
import functools

import jax
import jax.numpy as jnp
from jax import lax
from jax.experimental import pallas as pl
from jax.experimental.pallas import tpu as pltpu
from jax.experimental.pallas import tpu_sc as plsc

_NC = 2
_NS = 16
_NW = _NC * _NS

_DIM = 32
_BATCH = 16384
_HIST = 200
_B = _BATCH * _HIST

_CHUNK = 512
_SPB = _BATCH // _CHUNK
_NUNIT = _HIST * _SPB
_PER_W = _NUNIT // _NW
_TPC = _CHUNK // 128


def _make_kernel():
  mesh = plsc.VectorSubcoreMesh(
      core_axis_name="c", subcore_axis_name="s",
      num_cores=_NC, num_subcores=_NS)

  @functools.partial(
      pl.kernel,
      out_type=jax.ShapeDtypeStruct((_B * _DIM,), jnp.float32),
      mesh=mesh,
      scratch_types=(
          [pltpu.VMEM((_CHUNK,), jnp.int32) for _ in range(2)]
          + [pltpu.VMEM((_CHUNK, _DIM), jnp.float32) for _ in range(2)]
          + [pltpu.VMEM((_CHUNK * _DIM,), jnp.float32) for _ in range(2)]
          + [pltpu.SemaphoreType.DMA] * 3
      ),
      compiler_params=pltpu.CompilerParams(
          use_tc_tiling_on_sc=False, needs_layout_passes=False),
  )
  def gather_kernel(xt_hbm, table_hbm, out_hbm, idx0, idx1, rows0, rows1,
                    tr0, tr1, sem_i, sem_g, sem_s):
    idx_bufs = (idx0, idx1)
    row_bufs = (rows0, rows1)
    tr_bufs = (tr0, tr1)

    wid = lax.axis_index("s") * _NC + lax.axis_index("c")
    base_u = wid * _PER_W
    lane = lax.iota(jnp.int32, 16)

    def unit_hs(i):
      u = base_u + i
      return u // _SPB, u % _SPB

    def issue_idx(i, p):
      h, s = unit_hs(i)
      pltpu.async_copy(
          xt_hbm.at[pl.ds(h * _BATCH + s * _CHUNK, _CHUNK)],
          idx_bufs[p], sem_i)

    def issue_gather(p):
      pltpu.async_copy(table_hbm.at[idx_bufs[p]], row_bufs[p], sem_g)

    def issue_store(i, p):
      h, s = unit_hs(i)
      for dt in range(_DIM // 8):
        off = ((h * (_DIM // 8) + dt) * (_BATCH // 128) + s * _TPC) * 1024
        pltpu.async_copy(
            tr_bufs[p].at[pl.ds(dt * (_TPC * 1024), _TPC * 1024)],
            out_hbm.at[pl.ds(off, _TPC * 1024)], sem_s)

    def wait_idx():
      pltpu.make_async_copy(
          xt_hbm.at[pl.ds(0, _CHUNK)], idx_bufs[0], sem_i).wait()

    def wait_gather():
      pltpu.make_async_copy(
          table_hbm.at[idx_bufs[0]], row_bufs[0], sem_g).wait()

    def wait_store():
      for dt in range(_DIM // 8):
        pltpu.make_async_copy(
            tr_bufs[0].at[pl.ds(0, _TPC * 1024)],
            out_hbm.at[pl.ds(0, _TPC * 1024)], sem_s).wait()

    def transpose_unit(p):
      rows = row_bufs[p]
      tr = tr_bufs[p]

      @plsc.parallel_loop(0, _TPC * _DIM, 1, unroll=4)
      def _(z):
        bl = z >> 5
        d = z & (_DIM - 1)
        doff = bl * 1024 + (d >> 3) * (_TPC * 1024) + (d & 7) * 128
        rowv = lane + (bl * 128)
        colv = jnp.full((16,), d, jnp.int32)
        for g in range(8):
          v = plsc.load_gather(rows, [rowv + g * 16, colv])
          tr[pl.ds(doff + 16 * g, 16)] = v

    issue_idx(0, 0)
    issue_idx(1, 1)
    wait_idx()
    issue_gather(0)

    def round_body(r, carry):
      for p in range(2):
        i = 2 * r + p
        @pl.when(i + 1 < _PER_W)
        def _():
          wait_idx()
          issue_gather(1 - p)

        wait_gather()

        @pl.when(i >= 2)
        def _():
          wait_store()

        transpose_unit(p)
        issue_store(i, p)

        @pl.when(i + 2 < _PER_W)
        def _():
          issue_idx(i + 2, p)
      return carry

    lax.fori_loop(0, _PER_W // 2, round_body, 0)
    wait_store()
    wait_store()

  return gather_kernel


_GATHER = _make_kernel()


def kernel(x, table):
  xt = jnp.transpose(x).reshape(-1).astype(jnp.int32)
  u = _GATHER(xt, table)
  u5 = u.reshape(_HIST, _DIM // 8, _BATCH // 128, 8, 128)
  v = jnp.transpose(u5, (0, 1, 3, 2, 4)).reshape(_HIST, _DIM, _BATCH)
  return jnp.transpose(v, (2, 0, 1))


# speedup vs baseline: 2.8510x; 2.1462x over previous
import functools

import jax
import jax.numpy as jnp
from jax import lax
from jax.experimental import pallas as pl
from jax.experimental.pallas import tpu as pltpu
from jax.experimental.pallas import tpu_sc as plsc

_NC = 2
_NS = 16
_NW = _NC * _NS

_DIM = 32
_BATCH = 16384
_HIST = 200
_B = _BATCH * _HIST

_CHUNK = 512
_SPB = _BATCH // _CHUNK
_NUNIT = _HIST * _SPB
_PER_W = _NUNIT // _NW
_TPC = _CHUNK // 128


def _make_kernel():
  mesh = plsc.VectorSubcoreMesh(
      core_axis_name="c", subcore_axis_name="s",
      num_cores=_NC, num_subcores=_NS)

  @functools.partial(
      pl.kernel,
      out_type=jax.ShapeDtypeStruct((_B * _DIM,), jnp.float32),
      mesh=mesh,
      scratch_types=(
          [pltpu.VMEM((_CHUNK,), jnp.int32) for _ in range(2)]
          + [pltpu.VMEM((_CHUNK, _DIM), jnp.float32) for _ in range(2)]
          + [pltpu.VMEM((_CHUNK * _DIM,), jnp.float32) for _ in range(2)]
          + [pltpu.SemaphoreType.DMA] * 3
      ),
      compiler_params=pltpu.CompilerParams(
          use_tc_tiling_on_sc=False, needs_layout_passes=False),
  )
  def gather_kernel(xt_hbm, table_hbm, out_hbm, idx0, idx1, rows0, rows1,
                    tr0, tr1, sem_i, sem_g, sem_s):
    idx_bufs = (idx0, idx1)
    row_bufs = (rows0, rows1)
    tr_bufs = (tr0, tr1)

    wid = lax.axis_index("s") * _NC + lax.axis_index("c")
    base_u = wid * _PER_W
    lane = lax.iota(jnp.int32, 16)

    def unit_hs(i):
      u = base_u + i
      return u // _SPB, u % _SPB

    def issue_idx(i, p):
      h, s = unit_hs(i)
      pltpu.async_copy(
          xt_hbm.at[pl.ds(h * _BATCH + s * _CHUNK, _CHUNK)],
          idx_bufs[p], sem_i)

    def issue_gather(p):
      pltpu.async_copy(table_hbm.at[idx_bufs[p]], row_bufs[p], sem_g)

    def issue_store(i, p):
      h, s = unit_hs(i)
      for dt in range(_DIM // 8):
        off = ((h * (_DIM // 8) + dt) * (_BATCH // 128) + s * _TPC) * 1024
        pltpu.async_copy(
            tr_bufs[p].at[pl.ds(dt * (_TPC * 1024), _TPC * 1024)],
            out_hbm.at[pl.ds(off, _TPC * 1024)], sem_s)

    def wait_idx():
      pltpu.make_async_copy(
          xt_hbm.at[pl.ds(0, _CHUNK)], idx_bufs[0], sem_i).wait()

    def wait_gather():
      pltpu.make_async_copy(
          table_hbm.at[idx_bufs[0]], row_bufs[0], sem_g).wait()

    def wait_store():
      for dt in range(_DIM // 8):
        pltpu.make_async_copy(
            tr_bufs[0].at[pl.ds(0, _TPC * 1024)],
            out_hbm.at[pl.ds(0, _TPC * 1024)], sem_s).wait()

    def transpose_unit(p):
      rows = row_bufs[p]
      tr = tr_bufs[p]

      @plsc.parallel_loop(0, _DIM, 1, unroll=2)
      def _(d):
        colv = (lane + d) & (_DIM - 1)
        destv = ((colv >> 3) * (_TPC * 1024) + (colv & 7) * 128) + lane
        for bl in range(_TPC):
          for g in range(8):
            rowv = lane + (bl * 128 + 16 * g)
            v = plsc.load_gather(rows, [rowv, colv])
            plsc.store_scatter(tr, [destv + (bl * 1024 + 16 * g)], v)

    issue_idx(0, 0)
    issue_idx(1, 1)
    wait_idx()
    issue_gather(0)

    def round_body(r, carry):
      for p in range(2):
        i = 2 * r + p
        @pl.when(i + 1 < _PER_W)
        def _():
          wait_idx()
          issue_gather(1 - p)

        wait_gather()

        @pl.when(i >= 2)
        def _():
          wait_store()

        transpose_unit(p)
        issue_store(i, p)

        @pl.when(i + 2 < _PER_W)
        def _():
          issue_idx(i + 2, p)
      return carry

    lax.fori_loop(0, _PER_W // 2, round_body, 0)
    wait_store()
    wait_store()

  return gather_kernel


_GATHER = _make_kernel()


def kernel(x, table):
  xt = jnp.transpose(x).reshape(-1).astype(jnp.int32)
  u = _GATHER(xt, table)
  u5 = u.reshape(_HIST, _DIM // 8, _BATCH // 128, 8, 128)
  v = jnp.transpose(u5, (0, 1, 3, 2, 4)).reshape(_HIST, _DIM, _BATCH)
  return jnp.transpose(v, (2, 0, 1))


# speedup vs baseline: 2.8689x; 1.0063x over previous
import functools

import jax
import jax.numpy as jnp
from jax import lax
from jax.experimental import pallas as pl
from jax.experimental.pallas import tpu as pltpu
from jax.experimental.pallas import tpu_sc as plsc

_NC = 2
_NS = 16
_NW = _NC * _NS

_DIM = 32
_BATCH = 16384
_HIST = 200
_B = _BATCH * _HIST

_CHUNK = 512
_SPB = _BATCH // _CHUNK
_NUNIT = _HIST * _SPB
_PER_W = _NUNIT // _NW
_TPC = _CHUNK // 128


def _make_kernel():
  mesh = plsc.VectorSubcoreMesh(
      core_axis_name="c", subcore_axis_name="s",
      num_cores=_NC, num_subcores=_NS)

  @functools.partial(
      pl.kernel,
      out_type=jax.ShapeDtypeStruct((_B * _DIM,), jnp.float32),
      mesh=mesh,
      scratch_types=(
          [pltpu.VMEM((_CHUNK,), jnp.int32) for _ in range(2)]
          + [pltpu.VMEM((_CHUNK, _DIM), jnp.float32) for _ in range(2)]
          + [pltpu.VMEM((_CHUNK * _DIM,), jnp.float32) for _ in range(2)]
          + [pltpu.SemaphoreType.DMA] * 3
      ),
      compiler_params=pltpu.CompilerParams(
          use_tc_tiling_on_sc=False, needs_layout_passes=False),
  )
  def gather_kernel(xt_hbm, table_hbm, out_hbm, idx0, idx1, rows0, rows1,
                    tr0, tr1, sem_i, sem_g, sem_s):
    idx_bufs = (idx0, idx1)
    row_bufs = (rows0, rows1)
    tr_bufs = (tr0, tr1)

    wid = lax.axis_index("s") * _NC + lax.axis_index("c")
    base_u = wid * _PER_W
    lane = lax.iota(jnp.int32, 16)

    def unit_hs(i):
      u = base_u + i
      return u // _SPB, u % _SPB

    def issue_idx(i, p):
      h, s = unit_hs(i)
      pltpu.async_copy(
          xt_hbm.at[pl.ds(h * _BATCH + s * _CHUNK, _CHUNK)],
          idx_bufs[p], sem_i)

    def issue_gather(p):
      pltpu.async_copy(table_hbm.at[idx_bufs[p]], row_bufs[p], sem_g)

    def issue_store(i, p):
      h, s = unit_hs(i)
      for dt in range(_DIM // 8):
        off = ((h * (_DIM // 8) + dt) * (_BATCH // 128) + s * _TPC) * 1024
        pltpu.async_copy(
            tr_bufs[p].at[pl.ds(dt * (_TPC * 1024), _TPC * 1024)],
            out_hbm.at[pl.ds(off, _TPC * 1024)], sem_s)

    def wait_idx():
      pltpu.make_async_copy(
          xt_hbm.at[pl.ds(0, _CHUNK)], idx_bufs[0], sem_i).wait()

    def wait_gather():
      pltpu.make_async_copy(
          table_hbm.at[idx_bufs[0]], row_bufs[0], sem_g).wait()

    def wait_store():
      for dt in range(_DIM // 8):
        pltpu.make_async_copy(
            tr_bufs[0].at[pl.ds(0, _TPC * 1024)],
            out_hbm.at[pl.ds(0, _TPC * 1024)], sem_s).wait()

    def transpose_unit(p):
      rows = row_bufs[p]
      tr = tr_bufs[p]

      @plsc.parallel_loop(0, _DIM, 1, unroll=2)
      def _(d):
        colv = (lane + d) & (_DIM - 1)
        destv = ((colv >> 3) * (_TPC * 1024) + (colv & 7) * 128) + lane
        for bl in range(_TPC):
          for g in range(8):
            rowv = lane + (bl * 128 + 16 * g)
            v = plsc.load_gather(rows, [rowv, colv])
            plsc.store_scatter(tr, [destv + (bl * 1024 + 16 * g)], v)

    issue_idx(0, 0)
    issue_idx(1, 1)
    wait_idx()
    issue_gather(0)

    def round_body(r, carry):
      for p in range(2):
        i = 2 * r + p
        @pl.when(i + 1 < _PER_W)
        def _():
          wait_idx()
          issue_gather(1 - p)

        wait_gather()

        @pl.when(i >= 2)
        def _():
          wait_store()

        transpose_unit(p)
        issue_store(i, p)

        @pl.when(i + 2 < _PER_W)
        def _():
          issue_idx(i + 2, p)
      return carry

    lax.fori_loop(0, _PER_W // 2, round_body, 0)
    wait_store()
    wait_store()

  return gather_kernel


_GATHER = _make_kernel()


def kernel(x, table):
  xt = jnp.transpose(x).reshape(-1).astype(jnp.int32) * 4
  tpad = jnp.pad(table, ((0, 0), (0, 96))).reshape(4 * 1000000, _DIM)
  u = _GATHER(xt, tpad)
  u5 = u.reshape(_HIST, _DIM // 8, _BATCH // 128, 8, 128)
  v = jnp.transpose(u5, (0, 1, 3, 2, 4)).reshape(_HIST, _DIM, _BATCH)
  return jnp.transpose(v, (2, 0, 1))
